# baseline (speedup 1.0000x reference)
import functools
import math

import jax
import jax.numpy as jnp
from jax import lax
from jax.experimental import pallas as pl
from jax.experimental.pallas import tpu as pltpu
from jax.experimental.pallas import tpu_sc as plsc

_NUM_TOKEN_ID = 5
_EPS = 1e-12
_NBITS = 16
_BLK = 512
_SC_CHUNK = 128


def _sc_gather(word_emb, ids_flat):
    n = ids_flat.shape[0]
    h = word_emb.shape[1]
    info = plsc.get_sparse_core_info()
    nc, ns = info.num_cores, info.num_subcores
    nw = nc * ns
    bpw = n // nw
    n_chunks = bpw // _SC_CHUNK
    mesh = plsc.VectorSubcoreMesh(core_axis_name="c", subcore_axis_name="s")

    @functools.partial(
        pl.kernel,
        mesh=mesh,
        out_type=jax.ShapeDtypeStruct((n, h), jnp.float32),
        scratch_types=[
            pltpu.VMEM((bpw,), jnp.int32),
            pltpu.VMEM((_SC_CHUNK, h), jnp.float32),
            pltpu.SemaphoreType.DMA,
        ],
    )
    def k(table_hbm, idx_hbm, out_hbm, idx_v, rows_v, sem):
        wid = lax.axis_index("s") * nc + lax.axis_index("c")
        base = wid * bpw
        pltpu.sync_copy(idx_hbm.at[pl.ds(base, bpw)], idx_v)

        def body(j, carry):
            off = j * _SC_CHUNK
            pltpu.async_copy(
                table_hbm.at[idx_v.at[pl.ds(off, _SC_CHUNK)]], rows_v, sem
            ).wait()
            pltpu.sync_copy(rows_v, out_hbm.at[pl.ds(base + off, _SC_CHUNK)])
            return carry

        lax.fori_loop(0, n_chunks, body, 0)

    return k(word_emb, ids_flat)


def _ln(x, g, b):
    m = jnp.mean(x, axis=-1, keepdims=True)
    d = x - m
    v = jnp.mean(d * d, axis=-1, keepdims=True)
    return d * lax.rsqrt(v + _EPS) * g + b


def _fused_body(tw_ref, posw_hbm, ids_ref, vals_ref, fmt_ref, bin_ref,
                w1r_ref, w1bin_ref, b1_ref, w2_hbm, b2_ref,
                nlng_ref, nlnb_ref, gt_hbm, gn_hbm, gb_ref,
                lng_ref, lnb_ref, o_ref,
                acc_ref, posw_v, w2_v, gt_v, gn_v, sem):
    @pl.when(pl.program_id(0) == 0)
    def _():
        pltpu.make_async_copy(posw_hbm, posw_v, sem).start()
        pltpu.make_async_copy(posw_hbm, posw_v, sem).wait()
        pltpu.make_async_copy(w2_hbm, w2_v, sem).start()
        pltpu.make_async_copy(w2_hbm, w2_v, sem).wait()
        pltpu.make_async_copy(gt_hbm, gt_v, sem).start()
        pltpu.make_async_copy(gt_hbm, gt_v, sem).wait()
        pltpu.make_async_copy(gn_hbm, gn_v, sem).start()
        pltpu.make_async_copy(gn_hbm, gn_v, sem).wait()

    ids = ids_ref[...]
    text = tw_ref[...] + posw_v[...]
    acc_ref[...] = text
    has_num = jnp.any(ids == _NUM_TOKEN_ID)

    @pl.when(has_num)
    def _():
        v = vals_ref[...]
        av = jnp.abs(v)
        log_abs = jnp.log(av + 1e-6)
        sign = jnp.sign(v)
        expo = jnp.where(av > 1e-6,
                         jnp.floor(jnp.log(av + 1e-30) * (1.0 / math.log(10.0))),
                         0.0)
        fmt = fmt_ref[...]
        w1r = w1r_ref[...]
        h = (log_abs * w1r[0:1, :]
             + sign * w1r[1:2, :]
             + expo * w1r[2:3, :]
             + jnp.dot(bin_ref[...], w1bin_ref[...],
                       preferred_element_type=jnp.float32)
             + jnp.where(fmt == 0, 1.0, 0.0) * w1r[3:4, :]
             + jnp.where(fmt == 1, 1.0, 0.0) * w1r[4:5, :]
             + jnp.where(fmt == 2, 1.0, 0.0) * w1r[5:6, :]
             + b1_ref[...])
        h = 0.5 * h * (1.0 + lax.erf(h * (1.0 / math.sqrt(2.0))))
        h2 = jnp.dot(h, w2_v[...], preferred_element_type=jnp.float32) \
            + b2_ref[...]
        num = _ln(h2, nlng_ref[...], nlnb_ref[...])
        text_l = acc_ref[...]
        gate = jax.nn.sigmoid(
            jnp.dot(text_l, gt_v[...], preferred_element_type=jnp.float32)
            + jnp.dot(num, gn_v[...], preferred_element_type=jnp.float32)
            + gb_ref[...])
        fused = gate * num + (1.0 - gate) * text_l
        acc_ref[...] = jnp.where(ids == _NUM_TOKEN_ID, fused, text_l)

    o_ref[...] = _ln(acc_ref[...], lng_ref[...], lnb_ref[...])


def _fused_tc(tw, posw, ids_col, vals_col, fmt_col, binf,
              w1rows, w1bin, b1, w2, b2, nln_g, nln_b, gt, gn, gb, ln_g, ln_b):
    n, h = tw.shape
    inter = w2.shape[0]
    s = posw.shape[0]
    grid = n // _BLK
    full = lambda r, c: pl.BlockSpec((r, c), lambda i: (0, 0))
    blk = lambda c: pl.BlockSpec((_BLK, c), lambda i: (i, 0))
    anyspec = pl.BlockSpec(memory_space=pl.ANY)
    return pl.pallas_call(
        _fused_body,
        grid=(grid,),
        in_specs=[
            blk(h),
            anyspec,
            blk(1),
            blk(1),
            blk(1),
            blk(_NBITS),
            full(8, inter),
            full(_NBITS, inter),
            full(1, inter),
            anyspec,
            full(1, h),
            full(1, h), full(1, h),
            anyspec, anyspec,
            full(1, h),
            full(1, h), full(1, h),
        ],
        out_specs=blk(h),
        out_shape=jax.ShapeDtypeStruct((n, h), jnp.float32),
        scratch_shapes=[
            pltpu.VMEM((_BLK, h), jnp.float32),
            pltpu.VMEM((s, h), jnp.float32),
            pltpu.VMEM((inter, h), jnp.float32),
            pltpu.VMEM((h, h), jnp.float32),
            pltpu.VMEM((h, h), jnp.float32),
            pltpu.SemaphoreType.DMA,
        ],
    )(tw, posw, ids_col, vals_col, fmt_col, binf,
      w1rows, w1bin, b1, w2, b2, nln_g, nln_b, gt, gn, gb, ln_g, ln_b)


def kernel(input_ids, numeric_values, numeric_formats, word_emb, pos_emb,
           type_emb, ln_g, ln_b, w1, b1, w2, b2, nln_g, nln_b, gate_w, gate_b):
    b, s = input_ids.shape
    n = b * s
    hid = word_emb.shape[1]
    inter = w1.shape[1]

    ids_flat = input_ids.reshape(n).astype(jnp.int32)
    tw = _sc_gather(word_emb, ids_flat)

    posw = pos_emb[:s] + type_emb[0][None, :]
    ids_col = ids_flat.reshape(n, 1)
    vals_col = numeric_values.reshape(n, 1)
    fmt_col = numeric_formats.reshape(n, 1).astype(jnp.int32)
    binf = jax.random.normal(jax.random.key(1), (b, s, _NBITS),
                             dtype=jnp.float32).reshape(n, _NBITS)
    w1rows = jnp.concatenate(
        [w1[0:3], w1[3 + _NBITS:], jnp.zeros((2, inter), jnp.float32)], axis=0)
    w1bin = w1[3:3 + _NBITS]
    gt = gate_w[:hid]
    gn = gate_w[hid:]
    row = lambda x: x.reshape(1, -1)

    out = _fused_tc(tw, posw, ids_col, vals_col, fmt_col, binf,
                    w1rows, w1bin, row(b1), w2, row(b2),
                    row(nln_g), row(nln_b), gt, gn, row(gate_b),
                    row(ln_g), row(ln_b))
    return out.reshape(b, s, hid)


# speedup vs baseline: 1.5564x; 1.5564x over previous
import functools
import math

import jax
import jax.numpy as jnp
from jax import lax
from jax.experimental import pallas as pl
from jax.experimental.pallas import tpu as pltpu
from jax.experimental.pallas import tpu_sc as plsc

_NUM_TOKEN_ID = 5
_EPS = 1e-12
_NBITS = 16
_BLK = 512
_SC_CHUNK = 128


def _sc_gather(word_emb, ids_flat):
    n = ids_flat.shape[0]
    h = word_emb.shape[1]
    info = plsc.get_sparse_core_info()
    nc, ns = info.num_cores, info.num_subcores
    nw = nc * ns
    bpw = n // nw
    n_chunks = bpw // _SC_CHUNK
    mesh = plsc.VectorSubcoreMesh(core_axis_name="c", subcore_axis_name="s")

    @functools.partial(
        pl.kernel,
        mesh=mesh,
        out_type=jax.ShapeDtypeStruct((n, h), jnp.float32),
        scratch_types=[
            pltpu.VMEM((bpw,), jnp.int32),
            pltpu.VMEM((_SC_CHUNK, h), jnp.float32),
            pltpu.SemaphoreType.DMA,
        ],
    )
    def k(table_hbm, idx_hbm, out_hbm, idx_v, rows_v, sem):
        wid = lax.axis_index("s") * nc + lax.axis_index("c")
        base = wid * bpw
        pltpu.sync_copy(idx_hbm.at[pl.ds(base, bpw)], idx_v)

        def body(j, carry):
            off = j * _SC_CHUNK
            pltpu.async_copy(
                table_hbm.at[idx_v.at[pl.ds(off, _SC_CHUNK)]], rows_v, sem
            ).wait()
            pltpu.sync_copy(rows_v, out_hbm.at[pl.ds(base + off, _SC_CHUNK)])
            return carry

        lax.fori_loop(0, n_chunks, body, 0)

    return k(word_emb, ids_flat)


def _ln(x, g, b):
    m = jnp.mean(x, axis=-1, keepdims=True)
    d = x - m
    v = jnp.mean(d * d, axis=-1, keepdims=True)
    return d * lax.rsqrt(v + _EPS) * g + b


def _fused_body(tw_ref, posw_hbm, ids_ref, pc_ref,
                w1r_ref, w1bin_ref, b1_ref, w2_hbm, b2_ref,
                nlng_ref, nlnb_ref, gt_hbm, gn_hbm, gb_ref,
                lng_ref, lnb_ref, o_ref,
                acc_ref, posw_v, w2_v, gt_v, gn_v, sem):
    @pl.when(pl.program_id(0) == 0)
    def _():
        pltpu.make_async_copy(posw_hbm, posw_v, sem).start()
        pltpu.make_async_copy(posw_hbm, posw_v, sem).wait()
        pltpu.make_async_copy(w2_hbm, w2_v, sem).start()
        pltpu.make_async_copy(w2_hbm, w2_v, sem).wait()
        pltpu.make_async_copy(gt_hbm, gt_v, sem).start()
        pltpu.make_async_copy(gt_hbm, gt_v, sem).wait()
        pltpu.make_async_copy(gn_hbm, gn_v, sem).start()
        pltpu.make_async_copy(gn_hbm, gn_v, sem).wait()

    ids_row = ids_ref[0]
    text = tw_ref[...] + posw_v[...]
    acc_ref[...] = text
    has_num = jnp.any(ids_row == _NUM_TOKEN_ID)

    @pl.when(has_num)
    def _():
        pc = pc_ref[0]
        blkn = pc.shape[1]
        eye = jnp.asarray(
            lax.broadcasted_iota(jnp.int32, (blkn, blkn), 0)
            == lax.broadcasted_iota(jnp.int32, (blkn, blkn), 1),
            jnp.float32)
        cols = lax.dot_general(eye, pc, (((1,), (1,)), ((), ())),
                               preferred_element_type=jnp.float32)
        binv = cols[:, 0:_NBITS]
        v = cols[:, _NBITS:_NBITS + 1]
        fmt = cols[:, _NBITS + 1:_NBITS + 2]
        idf = cols[:, _NBITS + 2:_NBITS + 3]
        av = jnp.abs(v)
        log_abs = jnp.log(av + 1e-6)
        sign = jnp.sign(v)
        expo = jnp.where(av > 1e-6,
                         jnp.floor(jnp.log(av + 1e-30) * (1.0 / math.log(10.0))),
                         0.0)
        w1r = w1r_ref[...]
        h = (log_abs * w1r[0:1, :]
             + sign * w1r[1:2, :]
             + expo * w1r[2:3, :]
             + jnp.dot(binv, w1bin_ref[...],
                       preferred_element_type=jnp.float32)
             + jnp.where(fmt == 0.0, 1.0, 0.0) * w1r[3:4, :]
             + jnp.where(fmt == 1.0, 1.0, 0.0) * w1r[4:5, :]
             + jnp.where(fmt == 2.0, 1.0, 0.0) * w1r[5:6, :]
             + b1_ref[...])
        h = 0.5 * h * (1.0 + lax.erf(h * (1.0 / math.sqrt(2.0))))
        h2 = jnp.dot(h, w2_v[...], preferred_element_type=jnp.float32) \
            + b2_ref[...]
        num = _ln(h2, nlng_ref[...], nlnb_ref[...])
        text_l = acc_ref[...]
        gate = jax.nn.sigmoid(
            jnp.dot(text_l, gt_v[...], preferred_element_type=jnp.float32)
            + jnp.dot(num, gn_v[...], preferred_element_type=jnp.float32)
            + gb_ref[...])
        fused = gate * num + (1.0 - gate) * text_l
        acc_ref[...] = jnp.where(idf == float(_NUM_TOKEN_ID), fused, text_l)

    o_ref[...] = _ln(acc_ref[...], lng_ref[...], lnb_ref[...])


def _fused_tc(tw, posw, ids3, pcols,
              w1rows, w1bin, b1, w2, b2, nln_g, nln_b, gt, gn, gb, ln_g, ln_b):
    n, h = tw.shape
    inter = w2.shape[0]
    s = posw.shape[0]
    grid = n // _BLK
    ncols = pcols.shape[1]
    full = lambda r, c: pl.BlockSpec((r, c), lambda i: (0, 0))
    blk = lambda c: pl.BlockSpec((_BLK, c), lambda i: (i, 0))
    anyspec = pl.BlockSpec(memory_space=pl.ANY)
    return pl.pallas_call(
        _fused_body,
        grid=(grid,),
        in_specs=[
            blk(h),
            anyspec,
            pl.BlockSpec((1, 1, _BLK), lambda i: (i, 0, 0)),
            pl.BlockSpec((1, ncols, _BLK), lambda i: (i, 0, 0)),
            full(8, inter),
            full(_NBITS, inter),
            full(1, inter),
            anyspec,
            full(1, h),
            full(1, h), full(1, h),
            anyspec, anyspec,
            full(1, h),
            full(1, h), full(1, h),
        ],
        out_specs=blk(h),
        out_shape=jax.ShapeDtypeStruct((n, h), jnp.float32),
        scratch_shapes=[
            pltpu.VMEM((_BLK, h), jnp.float32),
            pltpu.VMEM((s, h), jnp.float32),
            pltpu.VMEM((inter, h), jnp.float32),
            pltpu.VMEM((h, h), jnp.float32),
            pltpu.VMEM((h, h), jnp.float32),
            pltpu.SemaphoreType.DMA,
        ],
    )(tw, posw, ids3, pcols,
      w1rows, w1bin, b1, w2, b2, nln_g, nln_b, gt, gn, gb, ln_g, ln_b)


def kernel(input_ids, numeric_values, numeric_formats, word_emb, pos_emb,
           type_emb, ln_g, ln_b, w1, b1, w2, b2, nln_g, nln_b, gate_w, gate_b):
    b, s = input_ids.shape
    n = b * s
    hid = word_emb.shape[1]
    inter = w1.shape[1]

    ids_flat = input_ids.reshape(n).astype(jnp.int32)
    tw = _sc_gather(word_emb, ids_flat)

    grid = n // _BLK
    posw = pos_emb[:s] + type_emb[0][None, :]
    ids3 = ids_flat.reshape(grid, 1, _BLK)
    binf = jax.random.normal(jax.random.key(1), (b, s, _NBITS),
                             dtype=jnp.float32).reshape(n, _NBITS)
    bin_t = binf.reshape(grid, _BLK, _NBITS).swapaxes(1, 2)
    pcols = jnp.concatenate([
        bin_t,
        numeric_values.reshape(grid, 1, _BLK),
        numeric_formats.astype(jnp.float32).reshape(grid, 1, _BLK),
        ids_flat.astype(jnp.float32).reshape(grid, 1, _BLK),
        jnp.zeros((grid, 5, _BLK), jnp.float32),
    ], axis=1)
    w1rows = jnp.concatenate(
        [w1[0:3], w1[3 + _NBITS:], jnp.zeros((2, inter), jnp.float32)], axis=0)
    w1bin = w1[3:3 + _NBITS]
    gt = gate_w[:hid]
    gn = gate_w[hid:]
    row = lambda x: x.reshape(1, -1)

    out = _fused_tc(tw, posw, ids3, pcols,
                    w1rows, w1bin, row(b1), w2, row(b2),
                    row(nln_g), row(nln_b), gt, gn, row(gate_b),
                    row(ln_g), row(ln_b))
    return out.reshape(b, s, hid)


# speedup vs baseline: 1.5889x; 1.0209x over previous
import functools
import math

import jax
import jax.numpy as jnp
from jax import lax
from jax.experimental import pallas as pl
from jax.experimental.pallas import tpu as pltpu
from jax.experimental.pallas import tpu_sc as plsc

_NUM_TOKEN_ID = 5
_EPS = 1e-12
_NBITS = 16
_BLK = 512
_SC_CHUNK = 64


def _sc_gather(word_emb, ids_flat):
    n = ids_flat.shape[0]
    h = word_emb.shape[1]
    info = plsc.get_sparse_core_info()
    nc, ns = info.num_cores, info.num_subcores
    nw = nc * ns
    bpw = n // nw
    n_chunks = bpw // _SC_CHUNK
    mesh = plsc.VectorSubcoreMesh(core_axis_name="c", subcore_axis_name="s")

    @functools.partial(
        pl.kernel,
        mesh=mesh,
        out_type=jax.ShapeDtypeStruct((n, h), jnp.float32),
        scratch_types=[
            pltpu.VMEM((bpw,), jnp.int32),
            pltpu.VMEM((_SC_CHUNK, h), jnp.float32),
            pltpu.VMEM((_SC_CHUNK, h), jnp.float32),
            pltpu.SemaphoreType.DMA,
            pltpu.SemaphoreType.DMA,
            pltpu.SemaphoreType.DMA,
            pltpu.SemaphoreType.DMA,
        ],
    )
    def k(table_hbm, idx_hbm, out_hbm, idx_v, rows0, rows1, g0, g1, s0, s1):
        wid = lax.axis_index("s") * nc + lax.axis_index("c")
        base = wid * bpw
        pltpu.sync_copy(idx_hbm.at[pl.ds(base, bpw)], idx_v)
        bufs = (rows0, rows1)
        gsems = (g0, g1)
        ssems = (s0, s1)

        def gather(j, b):
            pltpu.make_async_copy(
                table_hbm.at[idx_v.at[pl.ds(j * _SC_CHUNK, _SC_CHUNK)]],
                bufs[b], gsems[b]).start()

        gather(0, 0)
        gather(1, 1)

        def body(t, carry):
            for b in range(2):
                j = 2 * t + b
                pltpu.make_async_copy(
                    table_hbm.at[idx_v.at[pl.ds(j * _SC_CHUNK, _SC_CHUNK)]],
                    bufs[b], gsems[b]).wait()
                dst = out_hbm.at[pl.ds(base + j * _SC_CHUNK, _SC_CHUNK)]
                pltpu.make_async_copy(bufs[b], dst, ssems[b]).start()

                @pl.when(j + 2 < n_chunks)
                def _():
                    pltpu.make_async_copy(bufs[b], dst, ssems[b]).wait()
                    gather(j + 2, b)
            return carry

        lax.fori_loop(0, n_chunks // 2, body, 0)
        for b in range(2):
            j = n_chunks - 2 + b
            pltpu.make_async_copy(
                bufs[b],
                out_hbm.at[pl.ds(base + j * _SC_CHUNK, _SC_CHUNK)],
                ssems[b]).wait()

    return k(word_emb, ids_flat)


def _ln(x, g, b):
    m = jnp.mean(x, axis=-1, keepdims=True)
    d = x - m
    v = jnp.mean(d * d, axis=-1, keepdims=True)
    return d * lax.rsqrt(v + _EPS) * g + b


def _fused_body(tw_ref, posw_hbm, ids_ref, pc_ref,
                w1r_ref, w1bin_ref, b1_ref, w2_hbm, b2_ref,
                nlng_ref, nlnb_ref, gt_hbm, gn_hbm, gb_ref,
                lng_ref, lnb_ref, o_ref,
                acc_ref, posw_v, w2_v, gt_v, gn_v, sem):
    @pl.when(pl.program_id(0) == 0)
    def _():
        pltpu.make_async_copy(posw_hbm, posw_v, sem).start()
        pltpu.make_async_copy(posw_hbm, posw_v, sem).wait()
        pltpu.make_async_copy(w2_hbm, w2_v, sem).start()
        pltpu.make_async_copy(w2_hbm, w2_v, sem).wait()
        pltpu.make_async_copy(gt_hbm, gt_v, sem).start()
        pltpu.make_async_copy(gt_hbm, gt_v, sem).wait()
        pltpu.make_async_copy(gn_hbm, gn_v, sem).start()
        pltpu.make_async_copy(gn_hbm, gn_v, sem).wait()

    ids_row = ids_ref[0]
    text = tw_ref[...] + posw_v[...]
    acc_ref[...] = text
    has_num = jnp.any(ids_row == _NUM_TOKEN_ID)

    @pl.when(has_num)
    def _():
        pc = pc_ref[0]
        blkn = pc.shape[1]
        eye = jnp.asarray(
            lax.broadcasted_iota(jnp.int32, (blkn, blkn), 0)
            == lax.broadcasted_iota(jnp.int32, (blkn, blkn), 1),
            jnp.float32)
        cols = lax.dot_general(eye, pc, (((1,), (1,)), ((), ())),
                               preferred_element_type=jnp.float32)
        binv = cols[:, 0:_NBITS]
        v = cols[:, _NBITS:_NBITS + 1]
        fmt = cols[:, _NBITS + 1:_NBITS + 2]
        idf = cols[:, _NBITS + 2:_NBITS + 3]
        av = jnp.abs(v)
        log_abs = jnp.log(av + 1e-6)
        sign = jnp.sign(v)
        expo = jnp.where(av > 1e-6,
                         jnp.floor(jnp.log(av + 1e-30) * (1.0 / math.log(10.0))),
                         0.0)
        w1r = w1r_ref[...]
        h = (log_abs * w1r[0:1, :]
             + sign * w1r[1:2, :]
             + expo * w1r[2:3, :]
             + jnp.dot(binv, w1bin_ref[...],
                       preferred_element_type=jnp.float32)
             + jnp.where(fmt == 0.0, 1.0, 0.0) * w1r[3:4, :]
             + jnp.where(fmt == 1.0, 1.0, 0.0) * w1r[4:5, :]
             + jnp.where(fmt == 2.0, 1.0, 0.0) * w1r[5:6, :]
             + b1_ref[...])
        h = 0.5 * h * (1.0 + lax.erf(h * (1.0 / math.sqrt(2.0))))
        h2 = jnp.dot(h, w2_v[...], preferred_element_type=jnp.float32) \
            + b2_ref[...]
        num = _ln(h2, nlng_ref[...], nlnb_ref[...])
        text_l = acc_ref[...]
        gate = jax.nn.sigmoid(
            jnp.dot(text_l, gt_v[...], preferred_element_type=jnp.float32)
            + jnp.dot(num, gn_v[...], preferred_element_type=jnp.float32)
            + gb_ref[...])
        fused = gate * num + (1.0 - gate) * text_l
        acc_ref[...] = jnp.where(idf == float(_NUM_TOKEN_ID), fused, text_l)

    o_ref[...] = _ln(acc_ref[...], lng_ref[...], lnb_ref[...])


def _fused_tc(tw, posw, ids3, pcols,
              w1rows, w1bin, b1, w2, b2, nln_g, nln_b, gt, gn, gb, ln_g, ln_b):
    n, h = tw.shape
    inter = w2.shape[0]
    s = posw.shape[0]
    grid = n // _BLK
    ncols = pcols.shape[1]
    full = lambda r, c: pl.BlockSpec((r, c), lambda i: (0, 0))
    blk = lambda c: pl.BlockSpec((_BLK, c), lambda i: (i, 0))
    anyspec = pl.BlockSpec(memory_space=pl.ANY)
    return pl.pallas_call(
        _fused_body,
        grid=(grid,),
        in_specs=[
            blk(h),
            anyspec,
            pl.BlockSpec((1, 1, _BLK), lambda i: (i, 0, 0)),
            pl.BlockSpec((1, ncols, _BLK), lambda i: (i, 0, 0)),
            full(8, inter),
            full(_NBITS, inter),
            full(1, inter),
            anyspec,
            full(1, h),
            full(1, h), full(1, h),
            anyspec, anyspec,
            full(1, h),
            full(1, h), full(1, h),
        ],
        out_specs=blk(h),
        out_shape=jax.ShapeDtypeStruct((n, h), jnp.float32),
        scratch_shapes=[
            pltpu.VMEM((_BLK, h), jnp.float32),
            pltpu.VMEM((s, h), jnp.float32),
            pltpu.VMEM((inter, h), jnp.float32),
            pltpu.VMEM((h, h), jnp.float32),
            pltpu.VMEM((h, h), jnp.float32),
            pltpu.SemaphoreType.DMA,
        ],
    )(tw, posw, ids3, pcols,
      w1rows, w1bin, b1, w2, b2, nln_g, nln_b, gt, gn, gb, ln_g, ln_b)


def kernel(input_ids, numeric_values, numeric_formats, word_emb, pos_emb,
           type_emb, ln_g, ln_b, w1, b1, w2, b2, nln_g, nln_b, gate_w, gate_b):
    b, s = input_ids.shape
    n = b * s
    hid = word_emb.shape[1]
    inter = w1.shape[1]

    ids_flat = input_ids.reshape(n).astype(jnp.int32)
    tw = _sc_gather(word_emb, ids_flat)

    grid = n // _BLK
    posw = pos_emb[:s] + type_emb[0][None, :]
    ids3 = ids_flat.reshape(grid, 1, _BLK)
    binf = jax.random.normal(jax.random.key(1), (b, s, _NBITS),
                             dtype=jnp.float32).reshape(n, _NBITS)
    bin_t = binf.reshape(grid, _BLK, _NBITS).swapaxes(1, 2)
    pcols = jnp.concatenate([
        bin_t,
        numeric_values.reshape(grid, 1, _BLK),
        numeric_formats.astype(jnp.float32).reshape(grid, 1, _BLK),
        ids_flat.astype(jnp.float32).reshape(grid, 1, _BLK),
        jnp.zeros((grid, 5, _BLK), jnp.float32),
    ], axis=1)
    w1rows = jnp.concatenate(
        [w1[0:3], w1[3 + _NBITS:], jnp.zeros((2, inter), jnp.float32)], axis=0)
    w1bin = w1[3:3 + _NBITS]
    gt = gate_w[:hid]
    gn = gate_w[hid:]
    row = lambda x: x.reshape(1, -1)

    out = _fused_tc(tw, posw, ids3, pcols,
                    w1rows, w1bin, row(b1), w2, row(b2),
                    row(nln_g), row(nln_b), gt, gn, row(gate_b),
                    row(ln_g), row(ln_b))
    return out.reshape(b, s, hid)
